# Initial kernel scaffold; baseline (speedup 1.0000x reference)
import jax
import jax.numpy as jnp
from jax.experimental import pallas as pl


def kernel(heatmap, offset, wh):
    raise NotImplementedError("write your pallas kernel here")


# speedup vs baseline: 5.1356x; 5.1356x over previous
import functools

import jax
import jax.numpy as jnp
from jax.experimental import pallas as pl
from jax.experimental.pallas import tpu as pltpu

TOPK = 100
SCALE = 4.0


def _predict_kernel(hm_ref, off_ref, wh_ref, out_ref, hm_scr, rowmax_scr, *, C, H, W):
    NROWS = C * H
    x = hm_ref[0].reshape(NROWS, W)

    NEG = jnp.float32(-3.0e38)
    yc = jax.lax.broadcasted_iota(jnp.int32, (NROWS, W), 0) % H
    neg_row = jnp.full((1, W), NEG, jnp.float32)
    xm1 = jnp.concatenate([neg_row, x[:-1, :]], axis=0)
    xp1 = jnp.concatenate([x[1:, :], neg_row], axis=0)
    vmax = jnp.maximum(x, jnp.maximum(
        jnp.where(yc == 0, NEG, xm1), jnp.where(yc == H - 1, NEG, xp1)))
    neg_col = jnp.full((NROWS, 1), NEG, jnp.float32)
    hl = jnp.concatenate([neg_col, vmax[:, :-1]], axis=1)
    hr = jnp.concatenate([vmax[:, 1:], neg_col], axis=1)
    hmax = jnp.maximum(vmax, jnp.maximum(hl, hr))

    hm = jnp.where(hmax == x, x, jnp.float32(0.0))
    hm_scr[...] = hm
    rowmax_scr[...] = jnp.max(hm.reshape(C, H, W), axis=2)

    lane = jax.lax.broadcasted_iota(jnp.int32, (1, W), 1)
    lane_h = jax.lax.broadcasted_iota(jnp.int32, (1, H), 1)
    ridx = (jax.lax.broadcasted_iota(jnp.int32, (C, H), 0) * H
            + jax.lax.broadcasted_iota(jnp.int32, (C, H), 1))

    def round_body(k, carry):
        id_v, sc_v, x1_v, y1_v, x2_v, y2_v = carry
        rm = rowmax_scr[...]
        m = jnp.max(rm)
        r = jnp.min(jnp.where(rm == m, ridx, NROWS))
        row = hm_scr[pl.ds(r, 1), :]
        xq = jnp.min(jnp.where(row == m, lane, W))

        newrow = jnp.where(lane == xq, jnp.float32(-1.0), row)
        hm_scr[pl.ds(r, 1), :] = newrow
        nrm = jnp.max(newrow)
        c = r // H
        y = r - c * H
        rmrow = rowmax_scr[pl.ds(c, 1), :]
        rowmax_scr[pl.ds(c, 1), :] = jnp.where(lane_h == y, nrm, rmrow)

        sel = (lane == xq)
        offx = jnp.sum(jnp.where(sel, off_ref[0, 0, pl.ds(y, 1), :], 0.0))
        offy = jnp.sum(jnp.where(sel, off_ref[0, 1, pl.ds(y, 1), :], 0.0))
        bw = jnp.sum(jnp.where(sel, wh_ref[0, 0, pl.ds(y, 1), :], 0.0))
        bh = jnp.sum(jnp.where(sel, wh_ref[0, 1, pl.ds(y, 1), :], 0.0))

        xs = xq.astype(jnp.float32) + offx
        ys = y.astype(jnp.float32) + offy
        km = (lane == k)
        id_v = jnp.where(km, c.astype(jnp.float32), id_v)
        sc_v = jnp.where(km, m, sc_v)
        x1_v = jnp.where(km, (xs - 0.5 * bw) * SCALE, x1_v)
        y1_v = jnp.where(km, (ys - 0.5 * bh) * SCALE, y1_v)
        x2_v = jnp.where(km, (xs + 0.5 * bw) * SCALE, x2_v)
        y2_v = jnp.where(km, (ys + 0.5 * bh) * SCALE, y2_v)
        return (id_v, sc_v, x1_v, y1_v, x2_v, y2_v)

    z = jnp.zeros((1, W), jnp.float32)
    outs = jax.lax.fori_loop(0, TOPK, round_body, (z, z, z, z, z, z))
    out_ref[0] = jnp.concatenate(list(outs) + [z, z], axis=0)


def kernel(heatmap, offset, wh):
    B, C, H, W = heatmap.shape
    out = pl.pallas_call(
        functools.partial(_predict_kernel, C=C, H=H, W=W),
        grid=(B,),
        in_specs=[
            pl.BlockSpec((1, C, H, W), lambda b: (b, 0, 0, 0)),
            pl.BlockSpec((1, 2, H, W), lambda b: (b, 0, 0, 0)),
            pl.BlockSpec((1, 2, H, W), lambda b: (b, 0, 0, 0)),
        ],
        out_specs=pl.BlockSpec((1, 8, W), lambda b: (b, 0, 0)),
        out_shape=jax.ShapeDtypeStruct((B, 8, W), jnp.float32),
        scratch_shapes=[
            pltpu.VMEM((C * H, W), jnp.float32),
            pltpu.VMEM((C, H), jnp.float32),
        ],
    )(heatmap, offset, wh)

    ids = out[:, 0, :TOPK][:, :, None]
    scores = out[:, 1, :TOPK][:, :, None]
    bboxes = jnp.transpose(out[:, 2:6, :TOPK], (0, 2, 1))
    return (ids, scores, bboxes)


# speedup vs baseline: 5.1669x; 1.0061x over previous
import functools

import jax
import jax.numpy as jnp
from jax.experimental import pallas as pl
from jax.experimental.pallas import tpu as pltpu

TOPK = 100
SCALE = 4.0


def _predict_kernel(hm_ref, off_ref, wh_ref, out_ref, hm_scr, rowmax_scr,
                    *, B, C, H, W):
    NROWS = C * H
    s = pl.program_id(0)

    @pl.when(s < B)
    def _maxpool():
        x = hm_ref[0].reshape(NROWS, W)
        NEG = jnp.float32(-3.0e38)
        yc = jax.lax.broadcasted_iota(jnp.int32, (NROWS, W), 0) % H
        neg_row = jnp.full((1, W), NEG, jnp.float32)
        xm1 = jnp.concatenate([neg_row, x[:-1, :]], axis=0)
        xp1 = jnp.concatenate([x[1:, :], neg_row], axis=0)
        vmax = jnp.maximum(x, jnp.maximum(
            jnp.where(yc == 0, NEG, xm1), jnp.where(yc == H - 1, NEG, xp1)))
        neg_col = jnp.full((NROWS, 1), NEG, jnp.float32)
        hl = jnp.concatenate([neg_col, vmax[:, :-1]], axis=1)
        hr = jnp.concatenate([vmax[:, 1:], neg_col], axis=1)
        hmax = jnp.maximum(vmax, jnp.maximum(hl, hr))

        hm = jnp.where(hmax == x, x, jnp.float32(0.0))
        hm_scr[pl.ds(s, 1)] = hm[None]
        rowmax_scr[pl.ds(s, 1)] = jnp.max(hm.reshape(C, H, W), axis=2)[None]

    @pl.when(s == B)
    def _rounds():
        lane = jax.lax.broadcasted_iota(jnp.int32, (1, W), 1)
        lane_h = jax.lax.broadcasted_iota(jnp.int32, (1, H), 1)
        ridx = (jax.lax.broadcasted_iota(jnp.int32, (C, H), 0) * H
                + jax.lax.broadcasted_iota(jnp.int32, (C, H), 1))

        def one_batch(b, k, carry):
            id_v, sc_v, x1_v, y1_v, x2_v, y2_v = carry
            rm = rowmax_scr[b]
            m = jnp.max(rm)
            r = jnp.min(jnp.where(rm == m, ridx, NROWS))
            row = hm_scr[b, pl.ds(r, 1), :]
            xq = jnp.min(jnp.where(row == m, lane, W))

            newrow = jnp.where(lane == xq, jnp.float32(-1.0), row)
            hm_scr[b, pl.ds(r, 1), :] = newrow
            nrm = jnp.max(newrow)
            c = r // H
            y = r - c * H
            rmrow = rowmax_scr[b, pl.ds(c, 1), :]
            rowmax_scr[b, pl.ds(c, 1), :] = jnp.where(lane_h == y, nrm, rmrow)

            sel = (lane == xq)
            offx = jnp.sum(jnp.where(sel, off_ref[b, 0, pl.ds(y, 1), :], 0.0))
            offy = jnp.sum(jnp.where(sel, off_ref[b, 1, pl.ds(y, 1), :], 0.0))
            bw = jnp.sum(jnp.where(sel, wh_ref[b, 0, pl.ds(y, 1), :], 0.0))
            bh = jnp.sum(jnp.where(sel, wh_ref[b, 1, pl.ds(y, 1), :], 0.0))

            xs = xq.astype(jnp.float32) + offx
            ys = y.astype(jnp.float32) + offy
            km = (lane == k)
            id_v = jnp.where(km, c.astype(jnp.float32), id_v)
            sc_v = jnp.where(km, m, sc_v)
            x1_v = jnp.where(km, (xs - 0.5 * bw) * SCALE, x1_v)
            y1_v = jnp.where(km, (ys - 0.5 * bh) * SCALE, y1_v)
            x2_v = jnp.where(km, (xs + 0.5 * bw) * SCALE, x2_v)
            y2_v = jnp.where(km, (ys + 0.5 * bh) * SCALE, y2_v)
            return (id_v, sc_v, x1_v, y1_v, x2_v, y2_v)

        def round_body(k, carries):
            return tuple(one_batch(b, k, carries[b]) for b in range(B))

        z = jnp.zeros((1, W), jnp.float32)
        init = tuple((z, z, z, z, z, z) for _ in range(B))
        outs = jax.lax.fori_loop(0, TOPK, round_body, init)
        zz = jnp.zeros((2, W), jnp.float32)
        for b in range(B):
            out_ref[b] = jnp.concatenate(list(outs[b]) + [zz], axis=0)


def kernel(heatmap, offset, wh):
    B, C, H, W = heatmap.shape
    out = pl.pallas_call(
        functools.partial(_predict_kernel, B=B, C=C, H=H, W=W),
        grid=(B + 1,),
        in_specs=[
            pl.BlockSpec((1, C, H, W),
                         lambda s: (jnp.minimum(s, B - 1), 0, 0, 0)),
            pl.BlockSpec((B, 2, H, W), lambda s: (0, 0, 0, 0)),
            pl.BlockSpec((B, 2, H, W), lambda s: (0, 0, 0, 0)),
        ],
        out_specs=pl.BlockSpec((B, 8, W), lambda s: (0, 0, 0)),
        out_shape=jax.ShapeDtypeStruct((B, 8, W), jnp.float32),
        scratch_shapes=[
            pltpu.VMEM((B, C * H, W), jnp.float32),
            pltpu.VMEM((B, C, H), jnp.float32),
        ],
    )(heatmap, offset, wh)

    ids = out[:, 0, :TOPK][:, :, None]
    scores = out[:, 1, :TOPK][:, :, None]
    bboxes = jnp.transpose(out[:, 2:6, :TOPK], (0, 2, 1))
    return (ids, scores, bboxes)


# speedup vs baseline: 5.9039x; 1.1426x over previous
import functools

import jax
import jax.numpy as jnp
from jax.experimental import pallas as pl
from jax.experimental.pallas import tpu as pltpu

TOPK = 100
SCALE = 4.0


def _predict_kernel(hm_ref, off_ref, wh_ref, out_ref, *scrs, B, C, H, W):
    hm_scrs = scrs[:B]
    rowmax_scrs = scrs[B:]
    NROWS = C * H
    s = pl.program_id(0)

    for b in range(B):

        @pl.when(s == b)
        def _maxpool(b=b):
            x = hm_ref[0].reshape(NROWS, W)
            NEG = jnp.float32(-3.0e38)
            yc = jax.lax.broadcasted_iota(jnp.int32, (NROWS, W), 0) % H
            neg_row = jnp.full((1, W), NEG, jnp.float32)
            xm1 = jnp.concatenate([neg_row, x[:-1, :]], axis=0)
            xp1 = jnp.concatenate([x[1:, :], neg_row], axis=0)
            vmax = jnp.maximum(x, jnp.maximum(
                jnp.where(yc == 0, NEG, xm1),
                jnp.where(yc == H - 1, NEG, xp1)))
            neg_col = jnp.full((NROWS, 1), NEG, jnp.float32)
            hl = jnp.concatenate([neg_col, vmax[:, :-1]], axis=1)
            hr = jnp.concatenate([vmax[:, 1:], neg_col], axis=1)
            hmax = jnp.maximum(vmax, jnp.maximum(hl, hr))

            hm = jnp.where(hmax == x, x, jnp.float32(0.0))
            hm_scrs[b][...] = hm
            rowmax_scrs[b][...] = jnp.max(hm.reshape(C, H, W), axis=2)

    @pl.when(s == B)
    def _rounds():
        lane = jax.lax.broadcasted_iota(jnp.int32, (1, W), 1)
        lane_h = jax.lax.broadcasted_iota(jnp.int32, (1, H), 1)
        ridx = (jax.lax.broadcasted_iota(jnp.int32, (C, H), 0) * H
                + jax.lax.broadcasted_iota(jnp.int32, (C, H), 1))

        def one_batch(b, k, carry):
            id_v, sc_v, x1_v, y1_v, x2_v, y2_v = carry
            rm = rowmax_scrs[b][...]
            m = jnp.max(rm)
            r = jnp.min(jnp.where(rm == m, ridx, NROWS))
            row = hm_scrs[b][pl.ds(r, 1), :]
            xq = jnp.min(jnp.where(row == m, lane, W))

            newrow = jnp.where(lane == xq, jnp.float32(-1.0), row)
            hm_scrs[b][pl.ds(r, 1), :] = newrow
            nrm = jnp.max(newrow)
            c = r // H
            y = r - c * H
            rmrow = rowmax_scrs[b][pl.ds(c, 1), :]
            rowmax_scrs[b][pl.ds(c, 1), :] = jnp.where(lane_h == y, nrm, rmrow)

            sel = (lane == xq)
            offx = jnp.sum(jnp.where(sel, off_ref[b, 0, pl.ds(y, 1), :], 0.0))
            offy = jnp.sum(jnp.where(sel, off_ref[b, 1, pl.ds(y, 1), :], 0.0))
            bw = jnp.sum(jnp.where(sel, wh_ref[b, 0, pl.ds(y, 1), :], 0.0))
            bh = jnp.sum(jnp.where(sel, wh_ref[b, 1, pl.ds(y, 1), :], 0.0))

            xs = xq.astype(jnp.float32) + offx
            ys = y.astype(jnp.float32) + offy
            km = (lane == k)
            id_v = jnp.where(km, c.astype(jnp.float32), id_v)
            sc_v = jnp.where(km, m, sc_v)
            x1_v = jnp.where(km, (xs - 0.5 * bw) * SCALE, x1_v)
            y1_v = jnp.where(km, (ys - 0.5 * bh) * SCALE, y1_v)
            x2_v = jnp.where(km, (xs + 0.5 * bw) * SCALE, x2_v)
            y2_v = jnp.where(km, (ys + 0.5 * bh) * SCALE, y2_v)
            return (id_v, sc_v, x1_v, y1_v, x2_v, y2_v)

        def round_body(k, carries):
            return tuple(one_batch(b, k, carries[b]) for b in range(B))

        z = jnp.zeros((1, W), jnp.float32)
        init = tuple((z, z, z, z, z, z) for _ in range(B))
        outs = jax.lax.fori_loop(0, TOPK, round_body, init)
        zz = jnp.zeros((2, W), jnp.float32)
        for b in range(B):
            out_ref[b] = jnp.concatenate(list(outs[b]) + [zz], axis=0)


def kernel(heatmap, offset, wh):
    B, C, H, W = heatmap.shape
    out = pl.pallas_call(
        functools.partial(_predict_kernel, B=B, C=C, H=H, W=W),
        grid=(B + 1,),
        in_specs=[
            pl.BlockSpec((1, C, H, W),
                         lambda s: (jnp.minimum(s, B - 1), 0, 0, 0)),
            pl.BlockSpec((B, 2, H, W), lambda s: (0, 0, 0, 0)),
            pl.BlockSpec((B, 2, H, W), lambda s: (0, 0, 0, 0)),
        ],
        out_specs=pl.BlockSpec((B, 8, W), lambda s: (0, 0, 0)),
        out_shape=jax.ShapeDtypeStruct((B, 8, W), jnp.float32),
        scratch_shapes=(
            [pltpu.VMEM((C * H, W), jnp.float32) for _ in range(B)]
            + [pltpu.VMEM((C, H), jnp.float32) for _ in range(B)]
        ),
    )(heatmap, offset, wh)

    ids = out[:, 0, :TOPK][:, :, None]
    scores = out[:, 1, :TOPK][:, :, None]
    bboxes = jnp.transpose(out[:, 2:6, :TOPK], (0, 2, 1))
    return (ids, scores, bboxes)


# speedup vs baseline: 17.3847x; 2.9446x over previous
import functools

import jax
import jax.numpy as jnp
from jax.experimental import pallas as pl
from jax.experimental.pallas import tpu as pltpu

TOPK = 100
SCALE = 4.0


def _predict_kernel(hm_ref, off_ref, wh_ref, out_ref, *scrs, B, C, H, W):
    hm_scrs = scrs[:B]
    rowmax_scrs = scrs[B:]
    NROWS = C * H
    s = pl.program_id(0)

    for b in range(B):

        @pl.when(s == b)
        def _maxpool(b=b):
            x = hm_ref[0].reshape(NROWS, W)
            NEG = jnp.float32(-3.0e38)
            yc = jax.lax.broadcasted_iota(jnp.int32, (NROWS, W), 0) % H
            neg_row = jnp.full((1, W), NEG, jnp.float32)
            xm1 = jnp.concatenate([neg_row, x[:-1, :]], axis=0)
            xp1 = jnp.concatenate([x[1:, :], neg_row], axis=0)
            vmax = jnp.maximum(x, jnp.maximum(
                jnp.where(yc == 0, NEG, xm1),
                jnp.where(yc == H - 1, NEG, xp1)))
            neg_col = jnp.full((NROWS, 1), NEG, jnp.float32)
            hl = jnp.concatenate([neg_col, vmax[:, :-1]], axis=1)
            hr = jnp.concatenate([vmax[:, 1:], neg_col], axis=1)
            hmax = jnp.maximum(vmax, jnp.maximum(hl, hr))

            hm = jnp.where(hmax == x, x, jnp.float32(0.0))
            hm_scrs[b][...] = hm
            rowmax_scrs[b][...] = jnp.max(hm.reshape(C, H, W), axis=2)

    @pl.when(s == B)
    def _rounds():
        lane = jax.lax.broadcasted_iota(jnp.int32, (1, W), 1)
        lane_h = jax.lax.broadcasted_iota(jnp.int32, (1, H), 1)
        ridx = (jax.lax.broadcasted_iota(jnp.int32, (C, H), 0) * H
                + jax.lax.broadcasted_iota(jnp.int32, (C, H), 1))
        R = range(B)

        def round_body(k, carries):
            rms = [rowmax_scrs[b][...] for b in R]
            ms = [jnp.max(rms[b]) for b in R]
            rs = [jnp.min(jnp.where(rms[b] == ms[b], ridx, NROWS)) for b in R]
            rows = [hm_scrs[b][pl.ds(rs[b], 1), :] for b in R]
            xqs = [jnp.min(jnp.where(rows[b] == ms[b], lane, W),
                           axis=1, keepdims=True) for b in R]
            newrows = [jnp.where(lane == xqs[b], jnp.float32(-1.0), rows[b])
                       for b in R]
            for b in R:
                hm_scrs[b][pl.ds(rs[b], 1), :] = newrows[b]
            nrms = [jnp.max(newrows[b], axis=1, keepdims=True) for b in R]
            cs = [rs[b] // H for b in R]
            ys = [rs[b] - cs[b] * H for b in R]
            rmrows = [rowmax_scrs[b][pl.ds(cs[b], 1), :] for b in R]
            for b in R:
                rowmax_scrs[b][pl.ds(cs[b], 1), :] = jnp.where(
                    lane_h == ys[b], nrms[b], rmrows[b])

            sels = [lane == xqs[b] for b in R]
            offxs = [jnp.sum(jnp.where(sels[b],
                                       off_ref[b, 0, pl.ds(ys[b], 1), :], 0.0),
                             axis=1, keepdims=True) for b in R]
            offys = [jnp.sum(jnp.where(sels[b],
                                       off_ref[b, 1, pl.ds(ys[b], 1), :], 0.0),
                             axis=1, keepdims=True) for b in R]
            bws = [jnp.sum(jnp.where(sels[b],
                                     wh_ref[b, 0, pl.ds(ys[b], 1), :], 0.0),
                           axis=1, keepdims=True) for b in R]
            bhs = [jnp.sum(jnp.where(sels[b],
                                     wh_ref[b, 1, pl.ds(ys[b], 1), :], 0.0),
                           axis=1, keepdims=True) for b in R]

            km = (lane == k)
            out = []
            for b in R:
                id_v, sc_v, x1_v, y1_v, x2_v, y2_v = carries[b]
                xs = xqs[b].astype(jnp.float32) + offxs[b]
                yv = ys[b].astype(jnp.float32) + offys[b]
                out.append((
                    jnp.where(km, cs[b].astype(jnp.float32), id_v),
                    jnp.where(km, ms[b], sc_v),
                    jnp.where(km, (xs - 0.5 * bws[b]) * SCALE, x1_v),
                    jnp.where(km, (yv - 0.5 * bhs[b]) * SCALE, y1_v),
                    jnp.where(km, (xs + 0.5 * bws[b]) * SCALE, x2_v),
                    jnp.where(km, (yv + 0.5 * bhs[b]) * SCALE, y2_v),
                ))
            return tuple(out)

        z = jnp.zeros((1, W), jnp.float32)
        init = tuple((z, z, z, z, z, z) for _ in range(B))
        outs = jax.lax.fori_loop(0, TOPK, round_body, init)
        zz = jnp.zeros((2, W), jnp.float32)
        for b in range(B):
            out_ref[b] = jnp.concatenate(list(outs[b]) + [zz], axis=0)


def kernel(heatmap, offset, wh):
    B, C, H, W = heatmap.shape
    out = pl.pallas_call(
        functools.partial(_predict_kernel, B=B, C=C, H=H, W=W),
        grid=(B + 1,),
        in_specs=[
            pl.BlockSpec((1, C, H, W),
                         lambda s: (jnp.minimum(s, B - 1), 0, 0, 0)),
            pl.BlockSpec((B, 2, H, W), lambda s: (0, 0, 0, 0)),
            pl.BlockSpec((B, 2, H, W), lambda s: (0, 0, 0, 0)),
        ],
        out_specs=pl.BlockSpec((B, 8, W), lambda s: (0, 0, 0)),
        out_shape=jax.ShapeDtypeStruct((B, 8, W), jnp.float32),
        scratch_shapes=(
            [pltpu.VMEM((C * H, W), jnp.float32) for _ in range(B)]
            + [pltpu.VMEM((C, H), jnp.float32) for _ in range(B)]
        ),
    )(heatmap, offset, wh)

    ids = out[:, 0, :TOPK][:, :, None]
    scores = out[:, 1, :TOPK][:, :, None]
    bboxes = jnp.transpose(out[:, 2:6, :TOPK], (0, 2, 1))
    return (ids, scores, bboxes)
